# baseline (speedup 1.0000x reference)
import jax
import jax.numpy as jnp
from jax import lax
from jax.experimental import pallas as pl
from jax.experimental.pallas import tpu as pltpu
from jax.experimental.pallas import tpu_sc as plsc

N = 10000
E = 320000
D_IN = 128
H = 256
HH = H // 2
OUT = 16
EPS = 1e-5

NC = 2
NS = 16
CHUNK = 80
EDGES_PER_WORKER = E // (NC * NS)
WT = 10
WROWS = N // WT
ZR = 40

C = 100
EROWS = E // C
RPT = EROWS // NS
BLK = 8
NBLK = RPT // BLK
DB = 8

_R = 2000
_G = N // _R


def _sc_mesh():
    return plsc.VectorSubcoreMesh(
        core_axis_name="c", subcore_axis_name="s", num_cores=NC, num_subcores=NS
    )


_DEG_BLOCKS = EROWS // DB


def _deg_body(dst2_hbm, zero_hbm, ones_hbm, deg_hbm,
              dst_blk, ones_v, stg, acc_sh, dsem):
    c = lax.axis_index("c")
    s = lax.axis_index("s")
    w = c * NS + s

    pltpu.sync_copy(ones_hbm, ones_v)

    @pl.when(s < WT)
    def _():
        pltpu.sync_copy(zero_hbm, stg)
        pltpu.sync_copy(stg, acc_sh.at[pl.ds(s * WROWS, WROWS)])

    plsc.subcore_barrier()

    nfull = _DEG_BLOCKS // (NC * NS)
    nextra = _DEG_BLOCKS - nfull * NC * NS
    nb = jnp.where(w < nextra, nfull + 1, nfull)

    def blk(t, carry):
        r0 = (w + NC * NS * t) * DB
        pltpu.sync_copy(dst2_hbm.at[pl.ds(r0, DB), :], dst_blk)
        ds_ = [pltpu.async_copy(ones_v, acc_sh.at[dst_blk.at[j]], dsem,
                                add=True)
               for j in range(DB)]
        for x in ds_:
            x.wait()
        return carry

    lax.fori_loop(0, nb, blk, 0)
    plsc.subcore_barrier()

    @pl.when(s < WT)
    def _():
        r0 = s * WROWS
        pltpu.sync_copy(acc_sh.at[pl.ds(r0, WROWS)], stg)
        pltpu.sync_copy(stg, deg_hbm.at[pl.ds(c * N + r0, WROWS)])


def _sc_degree(dst2):
    zero1 = jnp.zeros((WROWS,), jnp.float32)
    ones1 = jnp.ones((C,), jnp.float32)
    f = pl.kernel(
        _deg_body,
        out_type=jax.ShapeDtypeStruct((NC * N,), jnp.float32),
        mesh=_sc_mesh(),
        scratch_types=[
            pltpu.VMEM((DB, C), jnp.int32),
            pltpu.VMEM((C,), jnp.float32),
            pltpu.VMEM((WROWS,), jnp.float32),
            pltpu.VMEM_SHARED((N,), jnp.float32),
            pltpu.SemaphoreType.DMA,
        ],
        cost_estimate=pl.CostEstimate(
            flops=E, transcendentals=0, bytes_accessed=2 * 4 * E),
    )
    return f(dst2, zero1, ones1)


def _msg_body(hws_lo, hws_hi, src2_hbm, dst2_hbm, zrows_hbm,
              acc_lo, acc_hi, src_blk, dst_blk, src_blk2, dst_blk2,
              rows0, rows1, rows2,
              acc_sh, gs0, gs1, gs2, ss0, ss1, ss2, isem, isem2, isem3):
    c = lax.axis_index("c")
    s = lax.axis_index("s")

    def process(hws_hbm, acc_hbm):
        r0 = s * WROWS

        @pl.when(s < WT)
        def _():
            pltpu.sync_copy(zrows_hbm, rows0.at[pl.ds(0, ZR), :])
            zd = []
            for t in range(WROWS // ZR):
                zd.append(pltpu.async_copy(
                    rows0.at[pl.ds(0, ZR), :],
                    acc_sh.at[pl.ds(r0 + t * ZR, ZR), :], gs0))
            for x in zd:
                x.wait()

        plsc.subcore_barrier()

        row0 = s * RPT
        bufs = (rows0, rows1, rows2)
        gsems = (gs0, gs1, gs2)
        ssems = (ss0, ss1, ss2)
        NB = 3

        def run_pipeline(chunks, hook_at=None, hook=None):
            n = len(chunks)
            g = [None] * NB
            sc = [None] * NB
            for jj in range(min(2, n)):
                sb, _, j = chunks[jj]
                p = jj % NB
                g[p] = pltpu.async_copy(hws_hbm.at[sb.at[j]], bufs[p], gsems[p])
            for jj in range(n):
                p = jj % NB
                _, db, j = chunks[jj]
                g[p].wait()
                sc[p] = pltpu.async_copy(
                    bufs[p], acc_sh.at[db.at[j]], ssems[p], add=True)
                if jj + 2 < n:
                    if hook_at == jj:
                        hook()
                    q = (jj + 2) % NB
                    if sc[q] is not None:
                        sc[q].wait()
                    nsb, _, nj = chunks[jj + 2]
                    g[q] = pltpu.async_copy(
                        hws_hbm.at[nsb.at[nj]], bufs[q], gsems[q])
            for jj in range(max(0, n - NB), n):
                sc[jj % NB].wait()

        def pair_step(t, carry):
            r = row0 + t * 2 * BLK
            pltpu.sync_copy(src2_hbm.at[pl.ds(r, BLK), :], src_blk)
            dl = pltpu.async_copy(dst2_hbm.at[pl.ds(r, BLK), :], dst_blk, isem)
            sl2 = pltpu.async_copy(src2_hbm.at[pl.ds(r + BLK, BLK), :],
                                   src_blk2, isem2)
            dl2 = pltpu.async_copy(dst2_hbm.at[pl.ds(r + BLK, BLK), :],
                                   dst_blk2, isem3)
            dl.wait()
            chunks = ([(src_blk, dst_blk, j) for j in range(BLK)]
                      + [(src_blk2, dst_blk2, j) for j in range(BLK)])
            run_pipeline(chunks, hook_at=BLK - 2,
                         hook=lambda: (sl2.wait(), dl2.wait()))
            return carry

        lax.fori_loop(0, NBLK // 2, pair_step, 0)

        r = row0 + (NBLK // 2) * 2 * BLK
        pltpu.sync_copy(src2_hbm.at[pl.ds(r, BLK), :], src_blk)
        dl = pltpu.async_copy(dst2_hbm.at[pl.ds(r, BLK), :], dst_blk, isem)
        dl.wait()
        run_pipeline([(src_blk, dst_blk, j) for j in range(BLK)])
        plsc.subcore_barrier()

        @pl.when(s < WT)
        def _():
            d = [None, None]
            obufs = (rows0, rows1)
            osems = (ss0, ss1)
            for t in range(WROWS // ZR):
                p = t % 2
                r = r0 + t * ZR
                if d[p] is not None:
                    d[p].wait()
                pltpu.sync_copy(acc_sh.at[pl.ds(r, ZR), :],
                                obufs[p].at[pl.ds(0, ZR), :])
                d[p] = pltpu.async_copy(obufs[p].at[pl.ds(0, ZR), :],
                                        acc_hbm.at[pl.ds(r, ZR), :], osems[p])
            for x in d:
                if x is not None:
                    x.wait()

    @pl.when(c == 0)
    def _():
        process(hws_lo, acc_lo)

    @pl.when(c == 1)
    def _():
        process(hws_hi, acc_hi)


def _sc_message(hws_lo, hws_hi, src2, dst2):
    zrows = jnp.zeros((ZR, HH), jnp.float32)
    f = pl.kernel(
        _msg_body,
        out_type=(
            jax.ShapeDtypeStruct((N, HH), jnp.float32),
            jax.ShapeDtypeStruct((N, HH), jnp.float32),
        ),
        mesh=_sc_mesh(),
        scratch_types=[
            pltpu.VMEM((BLK, C), jnp.int32),
            pltpu.VMEM((BLK, C), jnp.int32),
            pltpu.VMEM((BLK, C), jnp.int32),
            pltpu.VMEM((BLK, C), jnp.int32),
            pltpu.VMEM((C, HH), jnp.float32),
            pltpu.VMEM((C, HH), jnp.float32),
            pltpu.VMEM((C, HH), jnp.float32),
            pltpu.VMEM_SHARED((N, HH), jnp.float32),
            pltpu.SemaphoreType.DMA,
            pltpu.SemaphoreType.DMA,
            pltpu.SemaphoreType.DMA,
            pltpu.SemaphoreType.DMA,
            pltpu.SemaphoreType.DMA,
            pltpu.SemaphoreType.DMA,
            pltpu.SemaphoreType.DMA,
            pltpu.SemaphoreType.DMA,
            pltpu.SemaphoreType.DMA,
        ],
        cost_estimate=pl.CostEstimate(
            flops=E * HH, transcendentals=0,
            bytes_accessed=2 * (E * HH * 4 + N * HH * 4)),
    )
    return f(hws_lo, hws_hi, src2, dst2, zrows)


def _tc1_body(x_ref, w_ref, d0_ref, d1_ref, lo_ref, hi_ref, dis_ref):
    deg = d0_ref[...] + d1_ref[...] + 1.0
    dis = lax.rsqrt(deg)
    hws = jnp.dot(x_ref[...], w_ref[...],
                  preferred_element_type=jnp.float32) * dis
    lo_ref[...] = hws[:, :HH]
    hi_ref[...] = hws[:, HH:]
    dis_ref[...] = dis


def _tc_layer1(x, W1, deg_p0, deg_p1):
    return pl.pallas_call(
        _tc1_body,
        grid=(_G,),
        in_specs=[
            pl.BlockSpec((_R, D_IN), lambda i: (i, 0)),
            pl.BlockSpec((D_IN, H), lambda i: (0, 0)),
            pl.BlockSpec((_R, 1), lambda i: (i, 0)),
            pl.BlockSpec((_R, 1), lambda i: (i, 0)),
        ],
        out_specs=[
            pl.BlockSpec((_R, HH), lambda i: (i, 0)),
            pl.BlockSpec((_R, HH), lambda i: (i, 0)),
            pl.BlockSpec((_R, 1), lambda i: (i, 0)),
        ],
        out_shape=[
            jax.ShapeDtypeStruct((N, HH), jnp.float32),
            jax.ShapeDtypeStruct((N, HH), jnp.float32),
            jax.ShapeDtypeStruct((N, 1), jnp.float32),
        ],
    )(x, W1, deg_p0, deg_p1)


_INV_STD = 1.0 / (1.0 + EPS) ** 0.5


def _act_halves(accl, acch, hwsl, hwsh, dis, b, g, be):
    outs = []
    for acc, hws, sl in ((accl, hwsl, slice(0, HH)), (acch, hwsh, slice(HH, H))):
        y = dis * (acc + hws) + b[:, sl]
        outs.append(jnp.maximum(y * (_INV_STD * g[:, sl]) + be[:, sl], 0.0))
    return outs


def _tcl_body(accl_ref, acch_ref, hwsl_ref, hwsh_ref, dis_ref,
              b_ref, g_ref, be_ref, w_ref, lo_ref, hi_ref):
    dis = dis_ref[...]
    al, ah = _act_halves(accl_ref[...], acch_ref[...], hwsl_ref[...],
                         hwsh_ref[...], dis, b_ref[...], g_ref[...], be_ref[...])
    w = w_ref[...]
    o = (jnp.dot(al, w[:HH, :], preferred_element_type=jnp.float32)
         + jnp.dot(ah, w[HH:, :], preferred_element_type=jnp.float32)) * dis
    lo_ref[...] = o[:, :HH]
    hi_ref[...] = o[:, HH:]


def _tc_layer(accl, acch, hwsl, hwsh, dis, b, g, be, Wn):
    return pl.pallas_call(
        _tcl_body,
        grid=(_G,),
        in_specs=[
            pl.BlockSpec((_R, HH), lambda i: (i, 0)),
            pl.BlockSpec((_R, HH), lambda i: (i, 0)),
            pl.BlockSpec((_R, HH), lambda i: (i, 0)),
            pl.BlockSpec((_R, HH), lambda i: (i, 0)),
            pl.BlockSpec((_R, 1), lambda i: (i, 0)),
            pl.BlockSpec((1, H), lambda i: (0, 0)),
            pl.BlockSpec((1, H), lambda i: (0, 0)),
            pl.BlockSpec((1, H), lambda i: (0, 0)),
            pl.BlockSpec((H, H), lambda i: (0, 0)),
        ],
        out_specs=[
            pl.BlockSpec((_R, HH), lambda i: (i, 0)),
            pl.BlockSpec((_R, HH), lambda i: (i, 0)),
        ],
        out_shape=[
            jax.ShapeDtypeStruct((N, HH), jnp.float32),
            jax.ShapeDtypeStruct((N, HH), jnp.float32),
        ],
    )(accl, acch, hwsl, hwsh, dis, b.reshape(1, H), g.reshape(1, H),
      be.reshape(1, H), Wn)


def _tcf_body(accl_ref, acch_ref, hwsl_ref, hwsh_ref, dis_ref,
              b_ref, g_ref, be_ref, wc1_ref, bc1_ref, wc2_ref, bc2_ref,
              out_ref, pool_ref):
    i = pl.program_id(0)
    al, ah = _act_halves(accl_ref[...], acch_ref[...], hwsl_ref[...],
                         hwsh_ref[...], dis_ref[...], b_ref[...], g_ref[...],
                         be_ref[...])
    part = jnp.concatenate(
        [jnp.sum(al, axis=0, keepdims=True),
         jnp.sum(ah, axis=0, keepdims=True)], axis=1)

    @pl.when(i == 0)
    def _():
        pool_ref[...] = part

    @pl.when(i > 0)
    def _():
        pool_ref[...] = pool_ref[...] + part

    @pl.when(i == _G - 1)
    def _():
        pooled = pool_ref[...] * (1.0 / N)
        z = jnp.maximum(
            jnp.dot(pooled, wc1_ref[...], preferred_element_type=jnp.float32)
            + bc1_ref[...], 0.0)
        out_ref[...] = (
            jnp.dot(z, wc2_ref[...], preferred_element_type=jnp.float32)
            + bc2_ref[...])


def _tc_final(accl, acch, hwsl, hwsh, dis, b, g, be, Wc1, bc1, Wc2, bc2):
    return pl.pallas_call(
        _tcf_body,
        grid=(_G,),
        in_specs=[
            pl.BlockSpec((_R, HH), lambda i: (i, 0)),
            pl.BlockSpec((_R, HH), lambda i: (i, 0)),
            pl.BlockSpec((_R, HH), lambda i: (i, 0)),
            pl.BlockSpec((_R, HH), lambda i: (i, 0)),
            pl.BlockSpec((_R, 1), lambda i: (i, 0)),
            pl.BlockSpec((1, H), lambda i: (0, 0)),
            pl.BlockSpec((1, H), lambda i: (0, 0)),
            pl.BlockSpec((1, H), lambda i: (0, 0)),
            pl.BlockSpec((H, HH), lambda i: (0, 0)),
            pl.BlockSpec((1, HH), lambda i: (0, 0)),
            pl.BlockSpec((HH, OUT), lambda i: (0, 0)),
            pl.BlockSpec((1, OUT), lambda i: (0, 0)),
        ],
        out_specs=pl.BlockSpec((1, OUT), lambda i: (0, 0)),
        out_shape=jax.ShapeDtypeStruct((1, OUT), jnp.float32),
        scratch_shapes=[pltpu.VMEM((1, H), jnp.float32)],
    )(accl, acch, hwsl, hwsh, dis, b.reshape(1, H), g.reshape(1, H),
      be.reshape(1, H), Wc1, bc1.reshape(1, HH), Wc2, bc2.reshape(1, OUT))


def kernel(x, edge_index, W1, b1, g1, be1, W2, b2, g2, be2,
           W3, b3, g3, be3, Wc1, bc1, Wc2, bc2):
    src = edge_index[0]
    dst = edge_index[1]
    src2 = src.reshape(EROWS, C)
    dst2 = dst.reshape(EROWS, C)

    deg_flat = _sc_degree(dst2)
    hws_lo, hws_hi, dis = _tc_layer1(
        x, W1, deg_flat[:N].reshape(N, 1), deg_flat[N:].reshape(N, 1))

    acc_lo, acc_hi = _sc_message(hws_lo, hws_hi, src2, dst2)
    hws_lo2, hws_hi2 = _tc_layer(acc_lo, acc_hi, hws_lo, hws_hi, dis,
                                 b1, g1, be1, W2)

    acc_lo2, acc_hi2 = _sc_message(hws_lo2, hws_hi2, src2, dst2)
    hws_lo3, hws_hi3 = _tc_layer(acc_lo2, acc_hi2, hws_lo2, hws_hi2, dis,
                                 b2, g2, be2, W3)

    acc_lo3, acc_hi3 = _sc_message(hws_lo3, hws_hi3, src2, dst2)
    return _tc_final(acc_lo3, acc_hi3, hws_lo3, hws_hi3, dis,
                     b3, g3, be3, Wc1, bc1, Wc2, bc2)


# speedup vs baseline: 1.0427x; 1.0427x over previous
import jax
import jax.numpy as jnp
from jax import lax
from jax.experimental import pallas as pl
from jax.experimental.pallas import tpu as pltpu
from jax.experimental.pallas import tpu_sc as plsc

N = 10000
E = 320000
D_IN = 128
H = 256
HH = H // 2
OUT = 16
EPS = 1e-5

NC = 2
NS = 16
CHUNK = 80
EDGES_PER_WORKER = E // (NC * NS)
WT = 10
WROWS = N // WT
ZR = 40

C = 100
EROWS = E // C
RPT = EROWS // NS
BLK = 8
NBLK = RPT // BLK
DB = 8

_R = 2000
_G = N // _R


def _sc_mesh():
    return plsc.VectorSubcoreMesh(
        core_axis_name="c", subcore_axis_name="s", num_cores=NC, num_subcores=NS
    )


_DEG_BLOCKS = EROWS // DB


def _deg_body(dst2_hbm, zero_hbm, ones_hbm, deg_hbm,
              dst_blk, ones_v, stg, acc_sh, dsem):
    c = lax.axis_index("c")
    s = lax.axis_index("s")
    w = c * NS + s

    pltpu.sync_copy(ones_hbm, ones_v)

    @pl.when(s < WT)
    def _():
        pltpu.sync_copy(zero_hbm, stg)
        pltpu.sync_copy(stg, acc_sh.at[pl.ds(s * WROWS, WROWS)])

    plsc.subcore_barrier()

    nfull = _DEG_BLOCKS // (NC * NS)
    nextra = _DEG_BLOCKS - nfull * NC * NS
    nb = jnp.where(w < nextra, nfull + 1, nfull)

    def blk(t, carry):
        r0 = (w + NC * NS * t) * DB
        pltpu.sync_copy(dst2_hbm.at[pl.ds(r0, DB), :], dst_blk)
        ds_ = [pltpu.async_copy(ones_v, acc_sh.at[dst_blk.at[j]], dsem,
                                add=True)
               for j in range(DB)]
        for x in ds_:
            x.wait()
        return carry

    lax.fori_loop(0, nb, blk, 0)
    plsc.subcore_barrier()

    @pl.when(s < WT)
    def _():
        r0 = s * WROWS
        pltpu.sync_copy(acc_sh.at[pl.ds(r0, WROWS)], stg)
        pltpu.sync_copy(stg, deg_hbm.at[pl.ds(c * N + r0, WROWS)])


def _sc_degree(dst2):
    zero1 = jnp.zeros((WROWS,), jnp.float32)
    ones1 = jnp.ones((C,), jnp.float32)
    f = pl.kernel(
        _deg_body,
        out_type=jax.ShapeDtypeStruct((NC * N,), jnp.float32),
        mesh=_sc_mesh(),
        scratch_types=[
            pltpu.VMEM((DB, C), jnp.int32),
            pltpu.VMEM((C,), jnp.float32),
            pltpu.VMEM((WROWS,), jnp.float32),
            pltpu.VMEM_SHARED((N,), jnp.float32),
            pltpu.SemaphoreType.DMA,
        ],
        cost_estimate=pl.CostEstimate(
            flops=E, transcendentals=0, bytes_accessed=2 * 4 * E),
    )
    return f(dst2, zero1, ones1)


def _msg_body(hws_lo, hws_hi, src2_hbm, dst2_hbm, zrows_hbm,
              acc_lo, acc_hi, src_blk, dst_blk, src_blk2, dst_blk2,
              src_blk3, dst_blk3, rows0, rows1, rows2,
              acc_sh, gs0, gs1, gs2, ss0, ss1, ss2,
              isem, isem2, isem3, isem4, isem5):
    c = lax.axis_index("c")
    s = lax.axis_index("s")

    def process(hws_hbm, acc_hbm):
        r0 = s * WROWS

        @pl.when(s < WT)
        def _():
            pltpu.sync_copy(zrows_hbm, rows0.at[pl.ds(0, ZR), :])
            zd = []
            for t in range(WROWS // ZR):
                zd.append(pltpu.async_copy(
                    rows0.at[pl.ds(0, ZR), :],
                    acc_sh.at[pl.ds(r0 + t * ZR, ZR), :], gs0))
            for x in zd:
                x.wait()

        plsc.subcore_barrier()

        row0 = s * RPT
        bufs = (rows0, rows1, rows2)
        gsems = (gs0, gs1, gs2)
        ssems = (ss0, ss1, ss2)
        NB = 3

        def run_pipeline(chunks, hooks=()):
            hooks = dict(hooks)
            n = len(chunks)
            g = [None] * NB
            sc = [None] * NB
            for jj in range(min(2, n)):
                sb, _, j = chunks[jj]
                p = jj % NB
                g[p] = pltpu.async_copy(hws_hbm.at[sb.at[j]], bufs[p], gsems[p])
            for jj in range(n):
                p = jj % NB
                _, db, j = chunks[jj]
                g[p].wait()
                sc[p] = pltpu.async_copy(
                    bufs[p], acc_sh.at[db.at[j]], ssems[p], add=True)
                if jj + 2 < n:
                    if jj in hooks:
                        hooks[jj]()
                    q = (jj + 2) % NB
                    if sc[q] is not None:
                        sc[q].wait()
                    nsb, _, nj = chunks[jj + 2]
                    g[q] = pltpu.async_copy(
                        hws_hbm.at[nsb.at[nj]], bufs[q], gsems[q])
            for jj in range(max(0, n - NB), n):
                sc[jj % NB].wait()

        iblks = ((src_blk, dst_blk, None, isem),
                 (src_blk2, dst_blk2, isem2, isem3),
                 (src_blk3, dst_blk3, isem4, isem5))
        TRI = 3

        def tri_step(t, carry):
            r = row0 + t * TRI * BLK
            pltpu.sync_copy(src2_hbm.at[pl.ds(r, BLK), :], src_blk)
            dl = pltpu.async_copy(dst2_hbm.at[pl.ds(r, BLK), :], dst_blk, isem)
            waits = {}
            chunks = [(src_blk, dst_blk, j) for j in range(BLK)]
            for u in range(1, TRI):
                sb, db, sm1, sm2 = iblks[u]
                sl_ = pltpu.async_copy(
                    src2_hbm.at[pl.ds(r + u * BLK, BLK), :], sb, sm1)
                dl_ = pltpu.async_copy(
                    dst2_hbm.at[pl.ds(r + u * BLK, BLK), :], db, sm2)
                waits[u * BLK - 2] = (
                    lambda a=sl_, b=dl_: (a.wait(), b.wait()))
                chunks += [(sb, db, j) for j in range(BLK)]
            dl.wait()
            run_pipeline(chunks, waits.items())
            return carry

        lax.fori_loop(0, RPT // (TRI * BLK), tri_step, 0)

        r = row0 + (RPT // (TRI * BLK)) * TRI * BLK
        pltpu.sync_copy(src2_hbm.at[pl.ds(r, BLK), :], src_blk)
        dl = pltpu.async_copy(dst2_hbm.at[pl.ds(r, BLK), :], dst_blk, isem)
        dl.wait()
        run_pipeline([(src_blk, dst_blk, j) for j in range(BLK)])
        plsc.subcore_barrier()

        @pl.when(s < WT)
        def _():
            d = [None, None]
            obufs = (rows0, rows1)
            osems = (ss0, ss1)
            for t in range(WROWS // ZR):
                p = t % 2
                r = r0 + t * ZR
                if d[p] is not None:
                    d[p].wait()
                pltpu.sync_copy(acc_sh.at[pl.ds(r, ZR), :],
                                obufs[p].at[pl.ds(0, ZR), :])
                d[p] = pltpu.async_copy(obufs[p].at[pl.ds(0, ZR), :],
                                        acc_hbm.at[pl.ds(r, ZR), :], osems[p])
            for x in d:
                if x is not None:
                    x.wait()

    @pl.when(c == 0)
    def _():
        process(hws_lo, acc_lo)

    @pl.when(c == 1)
    def _():
        process(hws_hi, acc_hi)


def _sc_message(hws_lo, hws_hi, src2, dst2):
    zrows = jnp.zeros((ZR, HH), jnp.float32)
    f = pl.kernel(
        _msg_body,
        out_type=(
            jax.ShapeDtypeStruct((N, HH), jnp.float32),
            jax.ShapeDtypeStruct((N, HH), jnp.float32),
        ),
        mesh=_sc_mesh(),
        scratch_types=[
            pltpu.VMEM((BLK, C), jnp.int32),
            pltpu.VMEM((BLK, C), jnp.int32),
            pltpu.VMEM((BLK, C), jnp.int32),
            pltpu.VMEM((BLK, C), jnp.int32),
            pltpu.VMEM((BLK, C), jnp.int32),
            pltpu.VMEM((BLK, C), jnp.int32),
            pltpu.VMEM((C, HH), jnp.float32),
            pltpu.VMEM((C, HH), jnp.float32),
            pltpu.VMEM((C, HH), jnp.float32),
            pltpu.VMEM_SHARED((N, HH), jnp.float32),
            pltpu.SemaphoreType.DMA,
            pltpu.SemaphoreType.DMA,
            pltpu.SemaphoreType.DMA,
            pltpu.SemaphoreType.DMA,
            pltpu.SemaphoreType.DMA,
            pltpu.SemaphoreType.DMA,
            pltpu.SemaphoreType.DMA,
            pltpu.SemaphoreType.DMA,
            pltpu.SemaphoreType.DMA,
            pltpu.SemaphoreType.DMA,
            pltpu.SemaphoreType.DMA,
        ],
        cost_estimate=pl.CostEstimate(
            flops=E * HH, transcendentals=0,
            bytes_accessed=2 * (E * HH * 4 + N * HH * 4)),
    )
    return f(hws_lo, hws_hi, src2, dst2, zrows)


def _tc1_body(x_ref, w_ref, d0_ref, d1_ref, lo_ref, hi_ref, dis_ref):
    deg = d0_ref[...] + d1_ref[...] + 1.0
    dis = lax.rsqrt(deg)
    hws = jnp.dot(x_ref[...], w_ref[...],
                  preferred_element_type=jnp.float32) * dis
    lo_ref[...] = hws[:, :HH]
    hi_ref[...] = hws[:, HH:]
    dis_ref[...] = dis


def _tc_layer1(x, W1, deg_p0, deg_p1):
    return pl.pallas_call(
        _tc1_body,
        grid=(_G,),
        in_specs=[
            pl.BlockSpec((_R, D_IN), lambda i: (i, 0)),
            pl.BlockSpec((D_IN, H), lambda i: (0, 0)),
            pl.BlockSpec((_R, 1), lambda i: (i, 0)),
            pl.BlockSpec((_R, 1), lambda i: (i, 0)),
        ],
        out_specs=[
            pl.BlockSpec((_R, HH), lambda i: (i, 0)),
            pl.BlockSpec((_R, HH), lambda i: (i, 0)),
            pl.BlockSpec((_R, 1), lambda i: (i, 0)),
        ],
        out_shape=[
            jax.ShapeDtypeStruct((N, HH), jnp.float32),
            jax.ShapeDtypeStruct((N, HH), jnp.float32),
            jax.ShapeDtypeStruct((N, 1), jnp.float32),
        ],
    )(x, W1, deg_p0, deg_p1)


_INV_STD = 1.0 / (1.0 + EPS) ** 0.5


def _act_halves(accl, acch, hwsl, hwsh, dis, b, g, be):
    outs = []
    for acc, hws, sl in ((accl, hwsl, slice(0, HH)), (acch, hwsh, slice(HH, H))):
        y = dis * (acc + hws) + b[:, sl]
        outs.append(jnp.maximum(y * (_INV_STD * g[:, sl]) + be[:, sl], 0.0))
    return outs


def _tcl_body(accl_ref, acch_ref, hwsl_ref, hwsh_ref, dis_ref,
              b_ref, g_ref, be_ref, w_ref, lo_ref, hi_ref):
    dis = dis_ref[...]
    al, ah = _act_halves(accl_ref[...], acch_ref[...], hwsl_ref[...],
                         hwsh_ref[...], dis, b_ref[...], g_ref[...], be_ref[...])
    w = w_ref[...]
    o = (jnp.dot(al, w[:HH, :], preferred_element_type=jnp.float32)
         + jnp.dot(ah, w[HH:, :], preferred_element_type=jnp.float32)) * dis
    lo_ref[...] = o[:, :HH]
    hi_ref[...] = o[:, HH:]


def _tc_layer(accl, acch, hwsl, hwsh, dis, b, g, be, Wn):
    return pl.pallas_call(
        _tcl_body,
        grid=(_G,),
        in_specs=[
            pl.BlockSpec((_R, HH), lambda i: (i, 0)),
            pl.BlockSpec((_R, HH), lambda i: (i, 0)),
            pl.BlockSpec((_R, HH), lambda i: (i, 0)),
            pl.BlockSpec((_R, HH), lambda i: (i, 0)),
            pl.BlockSpec((_R, 1), lambda i: (i, 0)),
            pl.BlockSpec((1, H), lambda i: (0, 0)),
            pl.BlockSpec((1, H), lambda i: (0, 0)),
            pl.BlockSpec((1, H), lambda i: (0, 0)),
            pl.BlockSpec((H, H), lambda i: (0, 0)),
        ],
        out_specs=[
            pl.BlockSpec((_R, HH), lambda i: (i, 0)),
            pl.BlockSpec((_R, HH), lambda i: (i, 0)),
        ],
        out_shape=[
            jax.ShapeDtypeStruct((N, HH), jnp.float32),
            jax.ShapeDtypeStruct((N, HH), jnp.float32),
        ],
    )(accl, acch, hwsl, hwsh, dis, b.reshape(1, H), g.reshape(1, H),
      be.reshape(1, H), Wn)


def _tcf_body(accl_ref, acch_ref, hwsl_ref, hwsh_ref, dis_ref,
              b_ref, g_ref, be_ref, wc1_ref, bc1_ref, wc2_ref, bc2_ref,
              out_ref, pool_ref):
    i = pl.program_id(0)
    al, ah = _act_halves(accl_ref[...], acch_ref[...], hwsl_ref[...],
                         hwsh_ref[...], dis_ref[...], b_ref[...], g_ref[...],
                         be_ref[...])
    part = jnp.concatenate(
        [jnp.sum(al, axis=0, keepdims=True),
         jnp.sum(ah, axis=0, keepdims=True)], axis=1)

    @pl.when(i == 0)
    def _():
        pool_ref[...] = part

    @pl.when(i > 0)
    def _():
        pool_ref[...] = pool_ref[...] + part

    @pl.when(i == _G - 1)
    def _():
        pooled = pool_ref[...] * (1.0 / N)
        z = jnp.maximum(
            jnp.dot(pooled, wc1_ref[...], preferred_element_type=jnp.float32)
            + bc1_ref[...], 0.0)
        out_ref[...] = (
            jnp.dot(z, wc2_ref[...], preferred_element_type=jnp.float32)
            + bc2_ref[...])


def _tc_final(accl, acch, hwsl, hwsh, dis, b, g, be, Wc1, bc1, Wc2, bc2):
    return pl.pallas_call(
        _tcf_body,
        grid=(_G,),
        in_specs=[
            pl.BlockSpec((_R, HH), lambda i: (i, 0)),
            pl.BlockSpec((_R, HH), lambda i: (i, 0)),
            pl.BlockSpec((_R, HH), lambda i: (i, 0)),
            pl.BlockSpec((_R, HH), lambda i: (i, 0)),
            pl.BlockSpec((_R, 1), lambda i: (i, 0)),
            pl.BlockSpec((1, H), lambda i: (0, 0)),
            pl.BlockSpec((1, H), lambda i: (0, 0)),
            pl.BlockSpec((1, H), lambda i: (0, 0)),
            pl.BlockSpec((H, HH), lambda i: (0, 0)),
            pl.BlockSpec((1, HH), lambda i: (0, 0)),
            pl.BlockSpec((HH, OUT), lambda i: (0, 0)),
            pl.BlockSpec((1, OUT), lambda i: (0, 0)),
        ],
        out_specs=pl.BlockSpec((1, OUT), lambda i: (0, 0)),
        out_shape=jax.ShapeDtypeStruct((1, OUT), jnp.float32),
        scratch_shapes=[pltpu.VMEM((1, H), jnp.float32)],
    )(accl, acch, hwsl, hwsh, dis, b.reshape(1, H), g.reshape(1, H),
      be.reshape(1, H), Wc1, bc1.reshape(1, HH), Wc2, bc2.reshape(1, OUT))


def kernel(x, edge_index, W1, b1, g1, be1, W2, b2, g2, be2,
           W3, b3, g3, be3, Wc1, bc1, Wc2, bc2):
    src = edge_index[0]
    dst = edge_index[1]
    src2 = src.reshape(EROWS, C)
    dst2 = dst.reshape(EROWS, C)

    deg_flat = _sc_degree(dst2)
    hws_lo, hws_hi, dis = _tc_layer1(
        x, W1, deg_flat[:N].reshape(N, 1), deg_flat[N:].reshape(N, 1))

    acc_lo, acc_hi = _sc_message(hws_lo, hws_hi, src2, dst2)
    hws_lo2, hws_hi2 = _tc_layer(acc_lo, acc_hi, hws_lo, hws_hi, dis,
                                 b1, g1, be1, W2)

    acc_lo2, acc_hi2 = _sc_message(hws_lo2, hws_hi2, src2, dst2)
    hws_lo3, hws_hi3 = _tc_layer(acc_lo2, acc_hi2, hws_lo2, hws_hi2, dis,
                                 b2, g2, be2, W3)

    acc_lo3, acc_hi3 = _sc_message(hws_lo3, hws_hi3, src2, dst2)
    return _tc_final(acc_lo3, acc_hi3, hws_lo3, hws_hi3, dis,
                     b3, g3, be3, Wc1, bc1, Wc2, bc2)
